# Initial kernel scaffold; baseline (speedup 1.0000x reference)
import functools

import jax
import jax.numpy as jnp
from jax import lax
from jax.experimental import pallas as pl
from jax.experimental.pallas import tpu as pltpu
from jax.experimental.pallas import tpu_sc as plsc

_L = 240
_MAGIC = 12582912.0
_BLK = 8192


def _sc_body(x_hbm, t_hbm, s_hbm, o_hbm, traw, sraw, tbl):
    pltpu.sync_copy(t_hbm, traw)
    pltpu.sync_copy(s_hbm, sraw)

    slv = sraw[...]
    sv = jnp.exp(slv)
    qv = _L / sv
    fac = sv / 15.0
    lane = lax.iota(jnp.int32, 16)
    zero = jnp.zeros((16,), jnp.float32)

    for r in range(16):
        tbl[pl.ds(16 * r, 16)] = zero

    carry = zero
    for r in range(15):
        t = traw[pl.ds(16 * r, 16)]
        m = jnp.broadcast_to(jnp.max(t), (16,))
        e = jnp.exp(t - m)
        den = jnp.broadcast_to(jnp.sum(e), (16,))
        p = e / den
        c = plsc.cumsum(p) + carry
        plsc.store_scatter(tbl, [lane + (241 + 16 * r)], c * fac)
        carry = jnp.broadcast_to(jnp.max(c), (16,))

    def block(inv, outv):
        @pl.loop(0, _BLK, step=16)
        def _(i):
            xv = inv[pl.ds(i, 16)]
            v = jnp.minimum(jnp.maximum(xv * qv, -240.0), 240.0)
            f = (v + (_MAGIC + 240.0)) - _MAGIC
            idx = f.astype(jnp.int32)
            outv[pl.ds(i, 16)] = plsc.load_gather(tbl, [idx])

    n = x_hbm.shape[0]
    pltpu.emit_pipeline(
        block,
        grid=(n // _BLK,),
        in_specs=[pl.BlockSpec((_BLK,), lambda i: (i,))],
        out_specs=[pl.BlockSpec((_BLK,), lambda i: (i,))],
        core_axis_name=("core", "subcore"),
        dimension_semantics=(pltpu.PARALLEL,),
    )(x_hbm, o_hbm)


def kernel(x, table, scale_log):
    shape = x.shape
    n = x.size
    xf = x.reshape((n,))
    tf = table.reshape((-1,))
    sf = jnp.full((16,), scale_log, dtype=jnp.float32)

    mesh = plsc.VectorSubcoreMesh(core_axis_name="core",
                                  subcore_axis_name="subcore")
    run = functools.partial(
        pl.kernel,
        mesh=mesh,
        out_type=jax.ShapeDtypeStruct((n,), jnp.float32),
        scratch_types=[
            pltpu.VMEM((240,), jnp.float32),
            pltpu.VMEM((16,), jnp.float32),
            pltpu.VMEM((512,), jnp.float32),
        ],
    )(_sc_body)
    out = run(xf, tf, sf)
    return out.reshape(shape)


# speedup vs baseline: 171.2554x; 171.2554x over previous
import dataclasses
import functools

import jax
import jax.numpy as jnp
from jax import lax
from jax.experimental import pallas as pl
from jax.experimental.pallas import tpu as pltpu
from jax.experimental.pallas import tpu_sc as plsc

_L = 240
_MAGIC = 12582912.0
_BLK = 8192


def _sc_body(x_hbm, t_hbm, s_hbm, o_hbm, traw, sraw, tbl):
    pltpu.sync_copy(t_hbm, traw)
    pltpu.sync_copy(s_hbm, sraw)

    slv = sraw[...]
    sv = jnp.exp(slv)
    qv = _L / sv
    fac = sv / 15.0
    lane = lax.iota(jnp.int32, 16)
    zero = jnp.zeros((16,), jnp.float32)

    for r in range(16):
        tbl[pl.ds(16 * r, 16)] = zero

    carry = zero
    for r in range(15):
        t = traw[pl.ds(16 * r, 16)]
        m = jnp.broadcast_to(jnp.max(t), (16,))
        e = jnp.exp(t - m)
        den = jnp.broadcast_to(jnp.sum(e), (16,))
        p = e / den
        c = plsc.cumsum(p) + carry
        plsc.store_scatter(tbl, [lane + (241 + 16 * r)], c * fac)
        carry = jnp.broadcast_to(jnp.max(c), (16,))

    def block(inv, outv):
        @pl.loop(0, _BLK, step=16)
        def _(i):
            xv = inv[pl.ds(i, 16)]
            v = jnp.minimum(jnp.maximum(xv * qv, -240.0), 240.0)
            f = (v + (_MAGIC + 240.0)) - _MAGIC
            idx = f.astype(jnp.int32)
            outv[pl.ds(i, 16)] = plsc.load_gather(tbl, [idx])

    n = x_hbm.shape[0]
    pltpu.emit_pipeline(
        block,
        grid=(n // _BLK,),
        in_specs=[pl.BlockSpec((_BLK,), lambda i: (i,))],
        out_specs=[pl.BlockSpec((_BLK,), lambda i: (i,))],
        core_axis_name=("core", "subcore"),
        dimension_semantics=(pltpu.PARALLEL,),
    )(x_hbm, o_hbm)


def kernel(x, table, scale_log):
    shape = x.shape
    n = x.size
    xf = x.reshape((n,))
    tf = table.reshape((-1,))
    sf = jnp.full((16,), scale_log, dtype=jnp.float32)

    mesh = plsc.VectorSubcoreMesh(core_axis_name="core",
                                  subcore_axis_name="subcore")
    cp = pltpu.CompilerParams()
    if "needs_layout_passes" in pltpu.CompilerParams.__dataclass_fields__:
        cp = dataclasses.replace(cp, needs_layout_passes=False)
    run = functools.partial(
        pl.kernel,
        mesh=mesh,
        compiler_params=cp,
        out_type=jax.ShapeDtypeStruct((n,), jnp.float32),
        scratch_types=[
            pltpu.VMEM((240,), jnp.float32),
            pltpu.VMEM((16,), jnp.float32),
            pltpu.VMEM((512,), jnp.float32),
        ],
    )(_sc_body)
    out = run(xf, tf, sf)
    return out.reshape(shape)


# speedup vs baseline: 339.1559x; 1.9804x over previous
import dataclasses
import functools

import jax
import jax.numpy as jnp
from jax import lax
from jax.experimental import pallas as pl
from jax.experimental.pallas import tpu as pltpu
from jax.experimental.pallas import tpu_sc as plsc

_L = 240
_MAGIC = 12582912.0
_BLK = 8192


def _sc_body(x_hbm, t_hbm, s_hbm, o_hbm, traw, sraw, tbl):
    pltpu.sync_copy(t_hbm, traw)
    pltpu.sync_copy(s_hbm, sraw)

    slv = sraw[...]
    sv = jnp.exp(slv)
    qv = _L / sv
    fac = sv / 15.0
    lane = lax.iota(jnp.int32, 16)
    zero = jnp.zeros((16,), jnp.float32)

    for r in range(16):
        tbl[pl.ds(16 * r, 16)] = zero

    carry = zero
    for r in range(15):
        t = traw[pl.ds(16 * r, 16)]
        m = jnp.broadcast_to(jnp.max(t), (16,))
        e = jnp.exp(t - m)
        den = jnp.broadcast_to(jnp.sum(e), (16,))
        p = e / den
        c = plsc.cumsum(p) + carry
        plsc.store_scatter(tbl, [lane + (241 + 16 * r)], c * fac)
        carry = jnp.broadcast_to(jnp.max(c), (16,))

    def block(inv, outv):
        @plsc.parallel_loop(0, _BLK, step=16, unroll=8)
        def _(i):
            xv = inv[pl.ds(i, 16)]
            v = jnp.minimum(jnp.maximum(xv * qv, -240.0), 240.0)
            f = (v + (_MAGIC + 240.0)) - _MAGIC
            idx = f.astype(jnp.int32)
            outv[pl.ds(i, 16)] = plsc.load_gather(tbl, [idx])

    n = x_hbm.shape[0]
    pltpu.emit_pipeline(
        block,
        grid=(n // _BLK,),
        in_specs=[pl.BlockSpec((_BLK,), lambda i: (i,))],
        out_specs=[pl.BlockSpec((_BLK,), lambda i: (i,))],
        core_axis_name=("core", "subcore"),
        dimension_semantics=(pltpu.PARALLEL,),
    )(x_hbm, o_hbm)


def kernel(x, table, scale_log):
    shape = x.shape
    n = x.size
    xf = x.reshape((n,))
    tf = table.reshape((-1,))
    sf = jnp.full((16,), scale_log, dtype=jnp.float32)

    mesh = plsc.VectorSubcoreMesh(core_axis_name="core",
                                  subcore_axis_name="subcore")
    cp = pltpu.CompilerParams()
    if "needs_layout_passes" in pltpu.CompilerParams.__dataclass_fields__:
        cp = dataclasses.replace(cp, needs_layout_passes=False)
    run = functools.partial(
        pl.kernel,
        mesh=mesh,
        compiler_params=cp,
        out_type=jax.ShapeDtypeStruct((n,), jnp.float32),
        scratch_types=[
            pltpu.VMEM((240,), jnp.float32),
            pltpu.VMEM((16,), jnp.float32),
            pltpu.VMEM((512,), jnp.float32),
        ],
    )(_sc_body)
    out = run(xf, tf, sf)
    return out.reshape(shape)
